# baseline (speedup 1.0000x reference)
import functools

import jax
import jax.numpy as jnp
from jax import lax
from jax.experimental import pallas as pl
from jax.experimental.pallas import tpu as pltpu
from jax.experimental.pallas import tpu_sc as plsc

V = 100000
D = 300
DP = 304
B = 1024
CTX = 20
MAX_NORM = 1.0

NC = 2
NS = 16
NW = NC * NS
R = B * CTX
ROWS_PER_W = R // NW
CHUNK = 128
NCHUNK = ROWS_PER_W // CHUNK


def _sc_gather(table, idx):
    mesh = plsc.VectorSubcoreMesh(
        core_axis_name="c", subcore_axis_name="s", num_cores=NC, num_subcores=NS
    )

    @functools.partial(
        pl.kernel,
        out_type=jax.ShapeDtypeStruct((R, DP), jnp.float32),
        mesh=mesh,
        scratch_types=[
            pltpu.VMEM((CHUNK,), jnp.int32),
            pltpu.VMEM((CHUNK, DP), jnp.float32),
            pltpu.VMEM((CHUNK, DP), jnp.float32),
            pltpu.SemaphoreType.DMA,
            pltpu.SemaphoreType.DMA,
        ],
        compiler_params=pltpu.CompilerParams(use_tc_tiling_on_sc=False),
    )
    def gather_kernel(table_hbm, idx_hbm, e_hbm, idx_v, rows_a, rows_b, sem_a, sem_b):
        wid = lax.axis_index("s") * NC + lax.axis_index("c")
        base = wid * ROWS_PER_W
        rows = (rows_a, rows_b)
        sems = (sem_a, sem_b)
        for i in range(NCHUNK):
            off = base + i * CHUNK
            pltpu.sync_copy(idx_hbm.at[pl.ds(off, CHUNK)], idx_v)
            cp = pltpu.async_copy(table_hbm.at[idx_v], rows[i % 2], sems[i % 2])
            cp.wait()
            pltpu.sync_copy(rows[i % 2], e_hbm.at[pl.ds(off, CHUNK)])

    return gather_kernel(table, idx)


def _pool_kernel(e_ref, x_ref):
    e = e_ref[...]
    ss = jnp.sum(e * e, axis=2, keepdims=True)
    norm = jnp.sqrt(ss)
    scale = jnp.minimum(MAX_NORM, 1.0 / jnp.maximum(norm, 1e-7))
    pooled = jnp.sum(e * scale, axis=1) * (1.0 / CTX)
    x_ref[...] = lax.slice(pooled, (0, 0), (pooled.shape[0], D))


def _tc_pool(e3):
    BB = 128
    return pl.pallas_call(
        _pool_kernel,
        grid=(B // BB,),
        in_specs=[pl.BlockSpec((BB, CTX, DP), lambda i: (i, 0, 0))],
        out_specs=pl.BlockSpec((BB, D), lambda i: (i, 0)),
        out_shape=jax.ShapeDtypeStruct((B, D), jnp.float32),
    )(e3)


def _matmul_kernel(x_ref, w_ref, b_ref, o_ref):
    acc = lax.dot_general(
        x_ref[...],
        w_ref[...],
        dimension_numbers=(((1,), (1,)), ((), ())),
        preferred_element_type=jnp.float32,
    )
    o_ref[...] = acc + b_ref[...]


def _tc_matmul(x, W, b2):
    TV = 2048
    nvt = pl.cdiv(V, TV)
    return pl.pallas_call(
        _matmul_kernel,
        grid=(nvt,),
        in_specs=[
            pl.BlockSpec((B, D), lambda j: (0, 0)),
            pl.BlockSpec((TV, D), lambda j: (j, 0)),
            pl.BlockSpec((1, TV), lambda j: (0, j)),
        ],
        out_specs=pl.BlockSpec((B, TV), lambda j: (0, j)),
        out_shape=jax.ShapeDtypeStruct((B, V), jnp.float32),
    )(x, W, b2)


def kernel(inputs, emb_table, W, b):
    idx = inputs.reshape(R).astype(jnp.int32)
    table_p = jnp.pad(emb_table, ((0, 0), (0, DP - D)))
    e = _sc_gather(table_p, idx)
    x = _tc_pool(e.reshape(B, CTX, DP))
    return _tc_matmul(x, W, b.reshape(1, V))


# speedup vs baseline: 1.1239x; 1.1239x over previous
import functools

import jax
import jax.numpy as jnp
from jax import lax
from jax.experimental import pallas as pl
from jax.experimental.pallas import tpu as pltpu
from jax.experimental.pallas import tpu_sc as plsc

V = 100000
D = 300
DP = 384
B = 1024
CTX = 20
MAX_NORM = 1.0

NC = 2
NS = 16
NW = NC * NS
R = B * CTX
ROWS_PER_W = R // NW
CHUNK = 128
NCHUNK = ROWS_PER_W // CHUNK


def _sc_gather(table, idx):
    mesh = plsc.VectorSubcoreMesh(
        core_axis_name="c", subcore_axis_name="s", num_cores=NC, num_subcores=NS
    )

    @functools.partial(
        pl.kernel,
        out_type=jax.ShapeDtypeStruct((R, DP), jnp.float32),
        mesh=mesh,
        scratch_types=[
            pltpu.VMEM((CHUNK,), jnp.int32),
            pltpu.VMEM((CHUNK, DP), jnp.float32),
            pltpu.VMEM((CHUNK, DP), jnp.float32),
            pltpu.SemaphoreType.DMA,
            pltpu.SemaphoreType.DMA,
        ],
    )
    def gather_kernel(table_hbm, idx_hbm, e_hbm, idx_v, rows_a, rows_b, sem_a, sem_b):
        wid = lax.axis_index("s") * NC + lax.axis_index("c")
        base = wid * ROWS_PER_W
        rows = (rows_a, rows_b)
        sems = (sem_a, sem_b)
        for i in range(NCHUNK):
            off = base + i * CHUNK
            pltpu.sync_copy(idx_hbm.at[pl.ds(off, CHUNK)], idx_v)
            cp = pltpu.async_copy(table_hbm.at[idx_v], rows[i % 2], sems[i % 2])
            cp.wait()
            pltpu.sync_copy(rows[i % 2], e_hbm.at[pl.ds(off, CHUNK)])

    return gather_kernel(table, idx)


def _pool_kernel(e_ref, x_ref):
    e = e_ref[...]
    ss = jnp.sum(e * e, axis=2, keepdims=True)
    norm = jnp.sqrt(ss)
    scale = jnp.minimum(MAX_NORM, 1.0 / jnp.maximum(norm, 1e-7))
    pooled = jnp.sum(e * scale, axis=1) * (1.0 / CTX)
    x_ref[...] = lax.slice(pooled, (0, 0), (pooled.shape[0], D))


def _tc_pool(e3):
    BB = 128
    return pl.pallas_call(
        _pool_kernel,
        grid=(B // BB,),
        in_specs=[pl.BlockSpec((BB, CTX, DP), lambda i: (i, 0, 0))],
        out_specs=pl.BlockSpec((BB, D), lambda i: (i, 0)),
        out_shape=jax.ShapeDtypeStruct((B, D), jnp.float32),
    )(e3)


def _matmul_kernel(x_ref, w_ref, b_ref, o_ref):
    acc = lax.dot_general(
        x_ref[...],
        w_ref[...],
        dimension_numbers=(((1,), (1,)), ((), ())),
        preferred_element_type=jnp.float32,
    )
    o_ref[...] = acc + b_ref[...]


def _tc_matmul(x, W, b2):
    TV = 2048
    nvt = pl.cdiv(V, TV)
    return pl.pallas_call(
        _matmul_kernel,
        grid=(nvt,),
        in_specs=[
            pl.BlockSpec((B, D), lambda j: (0, 0)),
            pl.BlockSpec((TV, D), lambda j: (j, 0)),
            pl.BlockSpec((1, TV), lambda j: (0, j)),
        ],
        out_specs=pl.BlockSpec((B, TV), lambda j: (0, j)),
        out_shape=jax.ShapeDtypeStruct((B, V), jnp.float32),
    )(x, W, b2)


def kernel(inputs, emb_table, W, b):
    idx = inputs.reshape(R).astype(jnp.int32)
    table_p = jnp.pad(emb_table, ((0, 0), (0, DP - D)))
    e = _sc_gather(table_p, idx)
    x = _tc_pool(e.reshape(B, CTX, DP))
    return _tc_matmul(x, W, b.reshape(1, V))


# speedup vs baseline: 3.7581x; 3.3438x over previous
import functools

import jax
import jax.numpy as jnp
from jax import lax
from jax.experimental import pallas as pl
from jax.experimental.pallas import tpu as pltpu
from jax.experimental.pallas import tpu_sc as plsc

V = 100000
D = 300
DP = 384
B = 1024
CTX = 20
MAX_NORM = 1.0

NC = 2
NS = 16
NW = NC * NS
R = B * CTX
ROWS_PER_W = R // NW
CHUNK = 128
NCHUNK = ROWS_PER_W // CHUNK

CV = 2048


def _transpose_kernel(t_ref, o_ref):
    t = t_ref[...]
    o_ref[:, 0:D] = t.T
    o_ref[:, D:DP] = jnp.zeros((o_ref.shape[0], DP - D), jnp.float32)


def _tc_transpose(tT):
    return pl.pallas_call(
        _transpose_kernel,
        grid=(pl.cdiv(V, CV),),
        in_specs=[pl.BlockSpec((D, CV), lambda j: (0, j))],
        out_specs=pl.BlockSpec((CV, DP), lambda j: (j, 0)),
        out_shape=jax.ShapeDtypeStruct((V, DP), jnp.float32),
    )(tT)


def _sc_gather(table, idx):
    mesh = plsc.VectorSubcoreMesh(
        core_axis_name="c", subcore_axis_name="s", num_cores=NC, num_subcores=NS
    )

    @functools.partial(
        pl.kernel,
        out_type=jax.ShapeDtypeStruct((R, DP), jnp.float32),
        mesh=mesh,
        scratch_types=[
            pltpu.VMEM((CHUNK,), jnp.int32),
            pltpu.VMEM((CHUNK, DP), jnp.float32),
            pltpu.VMEM((CHUNK, DP), jnp.float32),
            pltpu.SemaphoreType.DMA,
            pltpu.SemaphoreType.DMA,
        ],
    )
    def gather_kernel(table_hbm, idx_hbm, e_hbm, idx_v, rows_a, rows_b, sem_a, sem_b):
        wid = lax.axis_index("s") * NC + lax.axis_index("c")
        base = wid * ROWS_PER_W
        rows = (rows_a, rows_b)
        sems = (sem_a, sem_b)
        for i in range(NCHUNK):
            off = base + i * CHUNK
            pltpu.sync_copy(idx_hbm.at[pl.ds(off, CHUNK)], idx_v)
            cp = pltpu.async_copy(table_hbm.at[idx_v], rows[i % 2], sems[i % 2])
            cp.wait()
            pltpu.sync_copy(rows[i % 2], e_hbm.at[pl.ds(off, CHUNK)])

    return gather_kernel(table, idx)


def _pool_kernel(e_ref, x_ref):
    e = e_ref[...]
    ss = jnp.sum(e * e, axis=2, keepdims=True)
    norm = jnp.sqrt(ss)
    scale = jnp.minimum(MAX_NORM, 1.0 / jnp.maximum(norm, 1e-7))
    pooled = jnp.sum(e * scale, axis=1) * (1.0 / CTX)
    x_ref[...] = lax.slice(pooled, (0, 0), (pooled.shape[0], D)).T


def _tc_pool(e3):
    BB = 128
    return pl.pallas_call(
        _pool_kernel,
        grid=(B // BB,),
        in_specs=[pl.BlockSpec((BB, CTX, DP), lambda i: (i, 0, 0))],
        out_specs=pl.BlockSpec((D, BB), lambda i: (0, i)),
        out_shape=jax.ShapeDtypeStruct((D, B), jnp.float32),
    )(e3)


def _matmul_kernel(w_ref, x_ref, b_ref, o_ref):
    acc = lax.dot_general(
        w_ref[...],
        x_ref[...],
        dimension_numbers=(((0,), (0,)), ((), ())),
        preferred_element_type=jnp.float32,
    )
    o_ref[...] = acc + b_ref[...]


def _tc_matmul(wT, xT, bc):
    TV = 2048
    nvt = pl.cdiv(V, TV)
    return pl.pallas_call(
        _matmul_kernel,
        grid=(nvt,),
        in_specs=[
            pl.BlockSpec((D, TV), lambda j: (0, j)),
            pl.BlockSpec((D, B), lambda j: (0, 0)),
            pl.BlockSpec((TV, 1), lambda j: (j, 0)),
        ],
        out_specs=pl.BlockSpec((TV, B), lambda j: (j, 0)),
        out_shape=jax.ShapeDtypeStruct((V, B), jnp.float32),
    )(wT, xT, bc)


def kernel(inputs, emb_table, W, b):
    idx = inputs.reshape(R).astype(jnp.int32)
    table_p = _tc_transpose(emb_table.T)
    e = _sc_gather(table_p, idx)
    xT = _tc_pool(e.reshape(B, CTX, DP))
    outT = _tc_matmul(W.T, xT, b.reshape(V, 1))
    return outT.T


# speedup vs baseline: 4.0498x; 1.0776x over previous
import functools

import jax
import jax.numpy as jnp
from jax import lax
from jax.experimental import pallas as pl
from jax.experimental.pallas import tpu as pltpu
from jax.experimental.pallas import tpu_sc as plsc

V = 100000
D = 300
DP = 384
B = 1024
CTX = 20
MAX_NORM = 1.0

NC = 2
NS = 16
NW = NC * NS
R = B * CTX
ROWS_PER_W = R // NW
CHUNK = 128
NCHUNK = ROWS_PER_W // CHUNK

CV = 2048


def _transpose_kernel(t_ref, o_ref):
    t = t_ref[...]
    o_ref[:, 0:D] = t.T
    o_ref[:, D:DP] = jnp.zeros((o_ref.shape[0], DP - D), jnp.float32)


def _tc_transpose(tT):
    return pl.pallas_call(
        _transpose_kernel,
        grid=(pl.cdiv(V, CV),),
        in_specs=[pl.BlockSpec((D, CV), lambda j: (0, j))],
        out_specs=pl.BlockSpec((CV, DP), lambda j: (j, 0)),
        out_shape=jax.ShapeDtypeStruct((V, DP), jnp.float32),
    )(tT)


def _sc_gather(table, idx):
    mesh = plsc.VectorSubcoreMesh(
        core_axis_name="c", subcore_axis_name="s", num_cores=NC, num_subcores=NS
    )

    @functools.partial(
        pl.kernel,
        out_type=jax.ShapeDtypeStruct((R, DP), jnp.float32),
        mesh=mesh,
        scratch_types=[
            pltpu.VMEM((CHUNK,), jnp.int32),
            pltpu.VMEM((CHUNK, DP), jnp.float32),
            pltpu.VMEM((CHUNK, DP), jnp.float32),
            pltpu.SemaphoreType.DMA,
            pltpu.SemaphoreType.DMA,
        ],
    )
    def gather_kernel(table_hbm, idx_hbm, e_hbm, idx_v, rows_a, rows_b, sem_a, sem_b):
        wid = lax.axis_index("s") * NC + lax.axis_index("c")
        base = wid * ROWS_PER_W
        rows = (rows_a, rows_b)
        sems = (sem_a, sem_b)
        for i in range(NCHUNK):
            off = base + i * CHUNK
            pltpu.sync_copy(idx_hbm.at[pl.ds(off, CHUNK)], idx_v)
            cp = pltpu.async_copy(table_hbm.at[idx_v], rows[i % 2], sems[i % 2])
            cp.wait()
            pltpu.sync_copy(rows[i % 2], e_hbm.at[pl.ds(off, CHUNK)])

    return gather_kernel(table, idx)


def _pool_kernel(e_ref, x_ref):
    e = e_ref[...]
    ss = jnp.sum(e * e, axis=2, keepdims=True)
    norm = jnp.sqrt(ss)
    scale = jnp.minimum(MAX_NORM, 1.0 / jnp.maximum(norm, 1e-7))
    pooled = jnp.sum(e * scale, axis=1) * (1.0 / CTX)
    x_ref[...] = lax.slice(pooled, (0, 0), (pooled.shape[0], D)).T


def _tc_pool(e3):
    BB = 128
    return pl.pallas_call(
        _pool_kernel,
        grid=(B // BB,),
        in_specs=[pl.BlockSpec((BB, CTX, DP), lambda i: (i, 0, 0))],
        out_specs=pl.BlockSpec((D, BB), lambda i: (0, i)),
        out_shape=jax.ShapeDtypeStruct((D, B), jnp.float32),
    )(e3)


def _matmul_kernel(w_ref, x_ref, b_ref, o_ref):
    acc = lax.dot_general(
        w_ref[...],
        x_ref[...],
        dimension_numbers=(((0,), (0,)), ((), ())),
        preferred_element_type=jnp.float32,
    )
    o_ref[...] = acc + b_ref[...].T


def _tc_matmul(wT, xT, bc):
    TV = 2048
    nvt = pl.cdiv(V, TV)
    return pl.pallas_call(
        _matmul_kernel,
        grid=(nvt,),
        in_specs=[
            pl.BlockSpec((D, TV), lambda j: (0, j)),
            pl.BlockSpec((D, B), lambda j: (0, 0)),
            pl.BlockSpec((1, TV), lambda j: (0, j)),
        ],
        out_specs=pl.BlockSpec((TV, B), lambda j: (j, 0)),
        out_shape=jax.ShapeDtypeStruct((V, B), jnp.float32),
    )(wT, xT, bc)


def kernel(inputs, emb_table, W, b):
    idx = inputs.reshape(R).astype(jnp.int32)
    table_p = _tc_transpose(emb_table.T)
    e = _sc_gather(table_p, idx)
    xT = _tc_pool(e.reshape(B, CTX, DP))
    outT = _tc_matmul(W.T, xT, b.reshape(1, V))
    return outT.T
